# baseline (speedup 1.0000x reference)
import functools

import jax
import jax.numpy as jnp
from jax import lax
from jax.experimental import pallas as pl
from jax.experimental.pallas import tpu as pltpu
from jax.experimental.pallas import tpu_sc as plsc

N = 10000
F_IN = 128
HID = 128
C_OUT = 16
E = 320000

NC, NS = 2, 16
NW = NC * NS
CH = 128
EPAD = 327680
JP1 = EPAD // (NS * 2 * CH)
JP2 = EPAD // (NW * 2 * CH)
NPAD = 10112
RPW = NPAD // NS

BLK = 400
GRID = N // BLK


def _pack_edges(src, dst):
    srcp = jnp.concatenate([src, jnp.zeros((EPAD - E,), jnp.int32)])
    dstp = jnp.concatenate([dst, jnp.full((EPAD - E,), N, jnp.int32)])
    packed = srcp | (dstp << 16)
    return (packed.reshape(NS, JP1, 2, CH), packed.reshape(NW, JP2, 2, CH))


def _make_segsum(d, feature_split):
    mesh = plsc.VectorSubcoreMesh(core_axis_name="c", subcore_axis_name="s")
    jp = JP1 if feature_split else JP2
    ng = NS if feature_split else NW

    @functools.partial(
        pl.kernel,
        mesh=mesh,
        compiler_params=pltpu.CompilerParams(use_tc_tiling_on_sc=False),
        out_type=jax.ShapeDtypeStruct((NC, NPAD, d), jnp.float32),
        scratch_types=[
            pltpu.VMEM((jp, 2, CH), jnp.int32),
            pltpu.VMEM((jp, 2, CH), jnp.int32),
            pltpu.VMEM((jp, 2, CH), jnp.int32),
            pltpu.VMEM((CH, d), jnp.float32),
            pltpu.VMEM((CH, d), jnp.float32),
            pltpu.VMEM_SHARED((NPAD, d), jnp.float32),
            pltpu.SemaphoreType.DMA,
            pltpu.SemaphoreType.DMA,
        ],
    )
    def segsum(table, eg, zrows, out,
               e_v, src_v, dst_v, rows0, rows1, acc, sem0, sem1):
        c = lax.axis_index("c")
        s = lax.axis_index("s")
        g = s if feature_split else c * NS + s
        pltpu.sync_copy(eg.at[g], e_v)

        def decode(i, carry):
            for half in range(2):
                for k in range(CH // 16):
                    w = e_v[i, half, pl.ds(16 * k, 16)]
                    sv = jnp.bitwise_and(w, 0xFFFF)
                    if feature_split:
                        sv = sv * 2 + c
                    src_v[i, half, pl.ds(16 * k, 16)] = sv
                    dst_v[i, half, pl.ds(16 * k, 16)] = lax.shift_right_logical(w, 16)
            return carry

        lax.fori_loop(0, jp, decode, 0)

        def zero_body(r, carry):
            pltpu.sync_copy(zrows, acc.at[pl.ds(s * RPW + r * 8, 8)])
            return carry

        lax.fori_loop(0, RPW // 8, zero_body, 0)
        plsc.subcore_barrier()

        pltpu.async_copy(table.at[src_v.at[0, 0]], rows0, sem0)

        def pair(i, carry):
            iw = jnp.where(i + 1 >= jp, 0, i + 1)
            pltpu.make_async_copy(table.at[src_v.at[i, 0]], rows0, sem0).wait()
            pltpu.async_copy(table.at[src_v.at[i, 1]], rows1, sem1)
            pltpu.sync_copy(rows0, acc.at[dst_v.at[i, 0]], add=True)
            pltpu.make_async_copy(table.at[src_v.at[i, 1]], rows1, sem1).wait()
            pltpu.async_copy(table.at[src_v.at[iw, 0]], rows0, sem0)
            pltpu.sync_copy(rows1, acc.at[dst_v.at[i, 1]], add=True)
            return carry

        lax.fori_loop(0, jp, pair, 0)
        pltpu.make_async_copy(table.at[src_v.at[0, 0]], rows0, sem0).wait()
        plsc.subcore_barrier()
        pltpu.sync_copy(acc.at[pl.ds(s * RPW, RPW)], out.at[c, pl.ds(s * RPW, RPW)])

    return segsum


_segsum_h = _make_segsum(HID // 2, feature_split=True)
_segsum_c = _make_segsum(C_OUT, feature_split=False)


def _mm_bias_body(x_ref, w_ref, b_ref, o_ref):
    o_ref[...] = (
        jnp.dot(x_ref[...], w_ref[...], preferred_element_type=jnp.float32)
        + b_ref[...]
    )


def _layer1_body(p_ref, xr_ref, w1l_ref, w2l_ref, h_ref, g_ref):
    agg = jnp.concatenate([p_ref[0], p_ref[1]], axis=-1)
    h = jnp.maximum(
        jnp.dot(agg, w1l_ref[...], preferred_element_type=jnp.float32) + xr_ref[...],
        0.0,
    )
    h_ref[...] = h
    g_ref[...] = jnp.dot(h, w2l_ref[...], preferred_element_type=jnp.float32)


def _layer2_body(p_ref, h_ref, w2r_ref, b_ref, o_ref):
    o_ref[...] = (
        p_ref[0]
        + p_ref[1]
        + jnp.dot(h_ref[...], w2r_ref[...], preferred_element_type=jnp.float32)
        + b_ref[...]
    )


def kernel(x, edge_index, W1_l, b1_l, W1_r, W2_l, b2_l, W2_r):
    eg1, eg2 = _pack_edges(edge_index[0], edge_index[1])
    zrows_h = jnp.zeros((8, HID // 2), jnp.float32)
    zrows_c = jnp.zeros((8, C_OUT), jnp.float32)

    xr1 = pl.pallas_call(
        _mm_bias_body,
        grid=(GRID,),
        in_specs=[
            pl.BlockSpec((BLK, F_IN), lambda i: (i, 0)),
            pl.BlockSpec((F_IN, HID), lambda i: (0, 0)),
            pl.BlockSpec((1, HID), lambda i: (0, 0)),
        ],
        out_specs=pl.BlockSpec((BLK, HID), lambda i: (i, 0)),
        out_shape=jax.ShapeDtypeStruct((N, HID), jnp.float32),
    )(x, W1_r, b1_l.reshape(1, HID))

    p1 = _segsum_h(x.reshape(2 * N, HID // 2), eg1, zrows_h)

    h, g = pl.pallas_call(
        _layer1_body,
        grid=(GRID,),
        in_specs=[
            pl.BlockSpec((NC, BLK, HID // 2), lambda i: (0, i, 0)),
            pl.BlockSpec((BLK, HID), lambda i: (i, 0)),
            pl.BlockSpec((F_IN, HID), lambda i: (0, 0)),
            pl.BlockSpec((HID, C_OUT), lambda i: (0, 0)),
        ],
        out_specs=[
            pl.BlockSpec((BLK, HID), lambda i: (i, 0)),
            pl.BlockSpec((BLK, C_OUT), lambda i: (i, 0)),
        ],
        out_shape=[
            jax.ShapeDtypeStruct((N, HID), jnp.float32),
            jax.ShapeDtypeStruct((N, C_OUT), jnp.float32),
        ],
    )(p1, xr1, W1_l, W2_l)

    p2 = _segsum_c(g, eg2, zrows_c)

    out = pl.pallas_call(
        _layer2_body,
        grid=(GRID,),
        in_specs=[
            pl.BlockSpec((NC, BLK, C_OUT), lambda i: (0, i, 0)),
            pl.BlockSpec((BLK, HID), lambda i: (i, 0)),
            pl.BlockSpec((HID, C_OUT), lambda i: (0, 0)),
            pl.BlockSpec((1, C_OUT), lambda i: (0, 0)),
        ],
        out_specs=pl.BlockSpec((BLK, C_OUT), lambda i: (i, 0)),
        out_shape=jax.ShapeDtypeStruct((N, C_OUT), jnp.float32),
    )(p2, h, W2_r, b2_l.reshape(1, C_OUT))
    return out


# speedup vs baseline: 1.3041x; 1.3041x over previous
import functools

import jax
import jax.numpy as jnp
from jax import lax
from jax.experimental import pallas as pl
from jax.experimental.pallas import tpu as pltpu
from jax.experimental.pallas import tpu_sc as plsc

N = 10000
F_IN = 128
HID = 128
C_OUT = 16
E = 320000

NC, NS = 2, 16
NW = NC * NS
CH = 128
EPAD = 327680
JP1 = EPAD // (NS * 2 * CH)
JP2 = EPAD // (NW * 2 * CH)
NPAD = 10112
RPW = NPAD // NS

BLK = 400
GRID = N // BLK


def _pack_edges(src, dst):
    srcp = jnp.concatenate([src, jnp.zeros((EPAD - E,), jnp.int32)])
    dstp = jnp.concatenate([dst, jnp.full((EPAD - E,), N, jnp.int32)])
    packed = srcp | (dstp << 16)
    return (packed.reshape(NS, JP1, 2, CH), packed.reshape(NW, JP2, 2, CH))


def _make_segsum(d, feature_split):
    mesh = plsc.VectorSubcoreMesh(core_axis_name="c", subcore_axis_name="s")
    jp = JP1 if feature_split else JP2
    ng = NS if feature_split else NW

    @functools.partial(
        pl.kernel,
        mesh=mesh,
        compiler_params=pltpu.CompilerParams(use_tc_tiling_on_sc=False),
        out_type=jax.ShapeDtypeStruct((NC, NPAD, d), jnp.float32),
        scratch_types=[
            pltpu.VMEM((jp, 2, CH), jnp.int32),
            pltpu.VMEM((jp, 2, CH), jnp.int32),
            pltpu.VMEM((jp, 2, CH), jnp.int32),
            pltpu.VMEM((CH, d), jnp.float32),
            pltpu.VMEM((CH, d), jnp.float32),
            pltpu.VMEM_SHARED((NPAD, d), jnp.float32),
            pltpu.SemaphoreType.DMA,
            pltpu.SemaphoreType.DMA,
        ],
    )
    def segsum(table, eg, zrows, out,
               e_v, src_v, dst_v, rows0, rows1, acc, sem0, sem1):
        c = lax.axis_index("c")
        s = lax.axis_index("s")
        g = s if feature_split else c * NS + s
        pltpu.sync_copy(eg.at[g], e_v)

        def decode(i):
            for half in range(2):
                for k in range(CH // 16):
                    w = e_v[i, half, pl.ds(16 * k, 16)]
                    sv = jnp.bitwise_and(w, 0xFFFF)
                    if feature_split:
                        sv = sv * 2 + c
                    src_v[i, half, pl.ds(16 * k, 16)] = sv
                    dst_v[i, half, pl.ds(16 * k, 16)] = lax.shift_right_logical(w, 16)

        pltpu.sync_copy(zrows, acc.at[pl.ds(s * RPW, RPW)])
        plsc.subcore_barrier()

        decode(0)
        pltpu.async_copy(table.at[src_v.at[0, 0]], rows0, sem0)

        def pair(i, carry):
            iw = jnp.where(i + 1 >= jp, 0, i + 1)
            decode(iw)
            pltpu.make_async_copy(table.at[src_v.at[i, 0]], rows0, sem0).wait()
            pltpu.async_copy(table.at[src_v.at[i, 1]], rows1, sem1)
            pltpu.sync_copy(rows0, acc.at[dst_v.at[i, 0]], add=True)
            pltpu.make_async_copy(table.at[src_v.at[i, 1]], rows1, sem1).wait()
            pltpu.async_copy(table.at[src_v.at[iw, 0]], rows0, sem0)
            pltpu.sync_copy(rows1, acc.at[dst_v.at[i, 1]], add=True)
            return carry

        lax.fori_loop(0, jp, pair, 0)
        pltpu.make_async_copy(table.at[src_v.at[0, 0]], rows0, sem0).wait()
        plsc.subcore_barrier()
        pltpu.sync_copy(acc.at[pl.ds(s * RPW, RPW)], out.at[c, pl.ds(s * RPW, RPW)])

    return segsum


_segsum_h = _make_segsum(HID // 2, feature_split=True)
_segsum_c = _make_segsum(C_OUT, feature_split=False)


def _mm_bias_body(x_ref, w_ref, b_ref, o_ref):
    o_ref[...] = (
        jnp.dot(x_ref[...], w_ref[...], preferred_element_type=jnp.float32)
        + b_ref[...]
    )


def _layer1_body(p_ref, xr_ref, w1l_ref, w2l_ref, h_ref, g_ref):
    agg = jnp.concatenate([p_ref[0], p_ref[1]], axis=-1)
    h = jnp.maximum(
        jnp.dot(agg, w1l_ref[...], preferred_element_type=jnp.float32) + xr_ref[...],
        0.0,
    )
    h_ref[...] = h
    g_ref[...] = jnp.dot(h, w2l_ref[...], preferred_element_type=jnp.float32)


def _layer2_body(p_ref, h_ref, w2r_ref, b_ref, o_ref):
    o_ref[...] = (
        p_ref[0]
        + p_ref[1]
        + jnp.dot(h_ref[...], w2r_ref[...], preferred_element_type=jnp.float32)
        + b_ref[...]
    )


def kernel(x, edge_index, W1_l, b1_l, W1_r, W2_l, b2_l, W2_r):
    eg1, eg2 = _pack_edges(edge_index[0], edge_index[1])
    zrows_h = jnp.zeros((RPW, HID // 2), jnp.float32)
    zrows_c = jnp.zeros((RPW, C_OUT), jnp.float32)

    xr1 = pl.pallas_call(
        _mm_bias_body,
        grid=(GRID,),
        in_specs=[
            pl.BlockSpec((BLK, F_IN), lambda i: (i, 0)),
            pl.BlockSpec((F_IN, HID), lambda i: (0, 0)),
            pl.BlockSpec((1, HID), lambda i: (0, 0)),
        ],
        out_specs=pl.BlockSpec((BLK, HID), lambda i: (i, 0)),
        out_shape=jax.ShapeDtypeStruct((N, HID), jnp.float32),
    )(x, W1_r, b1_l.reshape(1, HID))

    p1 = _segsum_h(x.reshape(2 * N, HID // 2), eg1, zrows_h)

    h, g = pl.pallas_call(
        _layer1_body,
        grid=(GRID,),
        in_specs=[
            pl.BlockSpec((NC, BLK, HID // 2), lambda i: (0, i, 0)),
            pl.BlockSpec((BLK, HID), lambda i: (i, 0)),
            pl.BlockSpec((F_IN, HID), lambda i: (0, 0)),
            pl.BlockSpec((HID, C_OUT), lambda i: (0, 0)),
        ],
        out_specs=[
            pl.BlockSpec((BLK, HID), lambda i: (i, 0)),
            pl.BlockSpec((BLK, C_OUT), lambda i: (i, 0)),
        ],
        out_shape=[
            jax.ShapeDtypeStruct((N, HID), jnp.float32),
            jax.ShapeDtypeStruct((N, C_OUT), jnp.float32),
        ],
    )(p1, xr1, W1_l, W2_l)

    p2 = _segsum_c(g, eg2, zrows_c)

    out = pl.pallas_call(
        _layer2_body,
        grid=(GRID,),
        in_specs=[
            pl.BlockSpec((NC, BLK, C_OUT), lambda i: (0, i, 0)),
            pl.BlockSpec((BLK, HID), lambda i: (i, 0)),
            pl.BlockSpec((HID, C_OUT), lambda i: (0, 0)),
            pl.BlockSpec((1, C_OUT), lambda i: (0, 0)),
        ],
        out_specs=pl.BlockSpec((BLK, C_OUT), lambda i: (i, 0)),
        out_shape=jax.ShapeDtypeStruct((N, C_OUT), jnp.float32),
    )(p2, h, W2_r, b2_l.reshape(1, C_OUT))
    return out


# speedup vs baseline: 1.3560x; 1.0398x over previous
import functools

import jax
import jax.numpy as jnp
from jax import lax
from jax.experimental import pallas as pl
from jax.experimental.pallas import tpu as pltpu
from jax.experimental.pallas import tpu_sc as plsc

N = 10000
F_IN = 128
HID = 128
C_OUT = 16
E = 320000

NC, NS = 2, 16
NW = NC * NS
CH = 128
EPAD = 327680
JP1 = EPAD // (NS * 2 * CH)
JP2 = EPAD // (NW * 2 * CH)
NPAD = 10112
RPW = NPAD // NS

BLK = 400
GRID = N // BLK


def _pack_edges(src, dst):
    srcp = jnp.concatenate([src, jnp.zeros((EPAD - E,), jnp.int32)])
    dstp = jnp.concatenate([dst, jnp.full((EPAD - E,), N, jnp.int32)])
    packed = srcp | (dstp << 16)
    return (packed.reshape(NS, JP1, 2, CH), packed.reshape(NW, JP2, 2, CH))


def _make_segsum(d, feature_split):
    mesh = plsc.VectorSubcoreMesh(core_axis_name="c", subcore_axis_name="s")
    jp = JP1 if feature_split else JP2
    ng = NS if feature_split else NW

    @functools.partial(
        pl.kernel,
        mesh=mesh,
        compiler_params=pltpu.CompilerParams(use_tc_tiling_on_sc=False),
        out_type=jax.ShapeDtypeStruct((NC, NPAD, d), jnp.float32),
        scratch_types=[
            pltpu.VMEM((4, 2, CH), jnp.int32),
            pltpu.VMEM((4, 2, CH), jnp.int32),
            pltpu.VMEM((4, 2, CH), jnp.int32),
            [pltpu.VMEM((CH, d), jnp.float32)] * 4,
            pltpu.VMEM_SHARED((NPAD, d), jnp.float32),
            [pltpu.SemaphoreType.DMA] * 4,
            [pltpu.SemaphoreType.DMA] * 4,
            pltpu.SemaphoreType.DMA,
        ],
    )
    def segsum(table, eg, zrows, out,
               e_v, src_v, dst_v, rows, acc, gsem, ssem, esem):
        c = lax.axis_index("c")
        s = lax.axis_index("s")
        g = s if feature_split else c * NS + s

        def wrap(p):
            return jnp.where(p >= jp, p - jp, p)

        def estage(p):
            pltpu.async_copy(eg.at[g, p], e_v.at[p & 3], esem)

        def estage_wait():
            pltpu.make_async_copy(eg.at[g, 0], e_v.at[0], esem).wait()

        def decode(p):
            sl = p & 3
            for half in range(2):
                for k in range(CH // 16):
                    w = e_v[sl, half, pl.ds(16 * k, 16)]
                    sv = jnp.bitwise_and(w, 0xFFFF)
                    if feature_split:
                        sv = sv * 2 + c
                    src_v[sl, half, pl.ds(16 * k, 16)] = sv
                    dst_v[sl, half, pl.ds(16 * k, 16)] = lax.shift_right_logical(w, 16)

        pltpu.sync_copy(zrows, acc.at[pl.ds(s * RPW, RPW)])
        plsc.subcore_barrier()

        nq = jp // 2

        def gather(p, h, b):
            pltpu.async_copy(table.at[src_v.at[p & 3, h]], rows[b], gsem[b])

        def gather_wait(p, h, b):
            pltpu.make_async_copy(
                table.at[src_v.at[p & 3, h]], rows[b], gsem[b]).wait()

        def scat(p, h, b):
            pltpu.async_copy(rows[b], acc.at[dst_v.at[p & 3, h]], ssem[b], add=True)

        def scat_wait(p, h, b):
            pltpu.make_async_copy(rows[b], acc.at[dst_v.at[p & 3, h]], ssem[b]).wait()

        for p in range(2):
            pltpu.sync_copy(eg.at[g, p], e_v.at[p])
        estage(jnp.int32(2))
        estage(jnp.int32(3))
        decode(0)
        decode(1)
        for b in range(4):
            gather(b // 2, b % 2, b)

        def quad(q, carry):
            p0 = 2 * q
            n0 = wrap(p0 + 2)
            n1 = wrap(p0 + 3)
            estage_wait()
            estage_wait()
            decode(n0)
            decode(n1)
            estage(wrap(p0 + 4))
            estage(wrap(p0 + 5))
            for b in range(4):
                pb = p0 if b < 2 else p0 + 1
                gather_wait(pb, b % 2, b)
                scat(pb, b % 2, b)
            for b in range(4):
                pb = p0 if b < 2 else p0 + 1
                nb = n0 if b < 2 else n1
                scat_wait(pb, b % 2, b)
                gather(nb, b % 2, b)
            return carry

        lax.fori_loop(0, nq, quad, 0)
        estage_wait()
        estage_wait()
        for b in range(4):
            gather_wait(b // 2, b % 2, b)
        plsc.subcore_barrier()
        pltpu.sync_copy(acc.at[pl.ds(s * RPW, RPW)], out.at[c, pl.ds(s * RPW, RPW)])

    return segsum


_segsum_h = _make_segsum(HID // 2, feature_split=True)
_segsum_c = _make_segsum(C_OUT // 2, feature_split=True)


def _mm_bias_body(x_ref, w_ref, b_ref, o_ref):
    o_ref[...] = (
        jnp.dot(x_ref[...], w_ref[...], preferred_element_type=jnp.float32)
        + b_ref[...]
    )


def _layer1_body(p_ref, xr_ref, w1l_ref, w2l_ref, h_ref, g_ref):
    agg = jnp.concatenate([p_ref[0], p_ref[1]], axis=-1)
    h = jnp.maximum(
        jnp.dot(agg, w1l_ref[...], preferred_element_type=jnp.float32) + xr_ref[...],
        0.0,
    )
    h_ref[...] = h
    g_ref[...] = jnp.dot(h, w2l_ref[...], preferred_element_type=jnp.float32)


def _layer2_body(p_ref, h_ref, w2r_ref, b_ref, o_ref):
    o_ref[...] = (
        jnp.concatenate([p_ref[0], p_ref[1]], axis=-1)
        + jnp.dot(h_ref[...], w2r_ref[...], preferred_element_type=jnp.float32)
        + b_ref[...]
    )


def kernel(x, edge_index, W1_l, b1_l, W1_r, W2_l, b2_l, W2_r):
    eg1, eg2 = _pack_edges(edge_index[0], edge_index[1])
    zrows_h = jnp.zeros((RPW, HID // 2), jnp.float32)
    zrows_c = jnp.zeros((RPW, C_OUT // 2), jnp.float32)

    xr1 = pl.pallas_call(
        _mm_bias_body,
        grid=(GRID,),
        in_specs=[
            pl.BlockSpec((BLK, F_IN), lambda i: (i, 0)),
            pl.BlockSpec((F_IN, HID), lambda i: (0, 0)),
            pl.BlockSpec((1, HID), lambda i: (0, 0)),
        ],
        out_specs=pl.BlockSpec((BLK, HID), lambda i: (i, 0)),
        out_shape=jax.ShapeDtypeStruct((N, HID), jnp.float32),
    )(x, W1_r, b1_l.reshape(1, HID))

    p1 = _segsum_h(x.reshape(2 * N, HID // 2), eg1, zrows_h)

    h, g = pl.pallas_call(
        _layer1_body,
        grid=(GRID,),
        in_specs=[
            pl.BlockSpec((NC, BLK, HID // 2), lambda i: (0, i, 0)),
            pl.BlockSpec((BLK, HID), lambda i: (i, 0)),
            pl.BlockSpec((F_IN, HID), lambda i: (0, 0)),
            pl.BlockSpec((HID, C_OUT), lambda i: (0, 0)),
        ],
        out_specs=[
            pl.BlockSpec((BLK, HID), lambda i: (i, 0)),
            pl.BlockSpec((BLK, C_OUT), lambda i: (i, 0)),
        ],
        out_shape=[
            jax.ShapeDtypeStruct((N, HID), jnp.float32),
            jax.ShapeDtypeStruct((N, C_OUT), jnp.float32),
        ],
    )(p1, xr1, W1_l, W2_l)

    p2 = _segsum_c(g.reshape(2 * N, C_OUT // 2), eg1, zrows_c)

    out = pl.pallas_call(
        _layer2_body,
        grid=(GRID,),
        in_specs=[
            pl.BlockSpec((NC, BLK, C_OUT // 2), lambda i: (0, i, 0)),
            pl.BlockSpec((BLK, HID), lambda i: (i, 0)),
            pl.BlockSpec((HID, C_OUT), lambda i: (0, 0)),
            pl.BlockSpec((1, C_OUT), lambda i: (0, 0)),
        ],
        out_specs=pl.BlockSpec((BLK, C_OUT), lambda i: (i, 0)),
        out_shape=jax.ShapeDtypeStruct((N, C_OUT), jnp.float32),
    )(p2, h, W2_r, b2_l.reshape(1, C_OUT))
    return out


# speedup vs baseline: 1.5642x; 1.1535x over previous
import functools

import jax
import jax.numpy as jnp
from jax import lax
from jax.experimental import pallas as pl
from jax.experimental.pallas import tpu as pltpu
from jax.experimental.pallas import tpu_sc as plsc

N = 10000
F_IN = 128
HID = 128
C_OUT = 16
E = 320000

NC, NS = 2, 16
NW = NC * NS
CH = 128
JB = 79
EPW = JB * CH
EPAD = EPW * NW
NPAD = 10112
RPW = NPAD // NS

BLK = 400
GRID = N // BLK


def _make_segsum(d):
    mesh = plsc.VectorSubcoreMesh(core_axis_name="c", subcore_axis_name="s")

    @functools.partial(
        pl.kernel,
        mesh=mesh,
        compiler_params=pltpu.CompilerParams(use_tc_tiling_on_sc=(d >= 128)),
        out_type=jax.ShapeDtypeStruct((NC, NPAD, d), jnp.float32),
        scratch_types=[
            pltpu.VMEM((JB, CH), jnp.int32),
            pltpu.VMEM((JB, CH), jnp.int32),
            pltpu.VMEM((CH, d), jnp.float32),
            pltpu.VMEM_SHARED((NPAD, d), jnp.float32),
            pltpu.SemaphoreType.DMA,
        ],
    )
    def segsum(table, srcg, dstg, zrows, out, src_v, dst_v, rows_v, acc, sem):
        c = lax.axis_index("c")
        s = lax.axis_index("s")
        g = c * NS + s
        pltpu.sync_copy(srcg.at[g], src_v)
        pltpu.sync_copy(dstg.at[g], dst_v)
        pltpu.sync_copy(zrows, acc.at[pl.ds(s * RPW, RPW)])
        plsc.subcore_barrier()

        def body(j, carry):
            pltpu.async_copy(table.at[src_v.at[j]], rows_v, sem).wait()
            pltpu.sync_copy(rows_v, acc.at[dst_v.at[j]], add=True)
            return carry

        lax.fori_loop(0, JB, body, 0)
        plsc.subcore_barrier()
        pltpu.sync_copy(acc.at[pl.ds(s * RPW, RPW)], out.at[c, pl.ds(s * RPW, RPW)])

    return segsum


_segsum_h = _make_segsum(HID)
_segsum_c = _make_segsum(C_OUT)


def _layer1_body(p_ref, x_ref, w1r_ref, b1_ref, w1l_ref, w2l_ref, h_ref, g_ref):
    agg = p_ref[0] + p_ref[1]
    h = jnp.maximum(
        jnp.dot(agg, w1l_ref[...], preferred_element_type=jnp.float32)
        + jnp.dot(x_ref[...], w1r_ref[...], preferred_element_type=jnp.float32)
        + b1_ref[...],
        0.0,
    )
    h_ref[...] = h
    g_ref[...] = jnp.dot(h, w2l_ref[...], preferred_element_type=jnp.float32)


def _layer2_body(p_ref, h_ref, w2r_ref, b_ref, o_ref):
    o_ref[...] = (
        p_ref[0]
        + p_ref[1]
        + jnp.dot(h_ref[...], w2r_ref[...], preferred_element_type=jnp.float32)
        + b_ref[...]
    )


def kernel(x, edge_index, W1_l, b1_l, W1_r, W2_l, b2_l, W2_r):
    src = edge_index[0]
    dst = edge_index[1]
    pad = EPAD - E
    srcg = jnp.concatenate([src, jnp.zeros((pad,), jnp.int32)]).reshape(NW, JB, CH)
    dstg = jnp.concatenate([dst, jnp.full((pad,), N, jnp.int32)]).reshape(NW, JB, CH)
    zrows_h = jnp.zeros((RPW, HID), jnp.float32)
    zrows_c = jnp.zeros((RPW, C_OUT), jnp.float32)

    p1 = _segsum_h(x, srcg, dstg, zrows_h)

    h, g = pl.pallas_call(
        _layer1_body,
        grid=(GRID,),
        in_specs=[
            pl.BlockSpec((NC, BLK, HID), lambda i: (0, i, 0)),
            pl.BlockSpec((BLK, F_IN), lambda i: (i, 0)),
            pl.BlockSpec((F_IN, HID), lambda i: (0, 0)),
            pl.BlockSpec((1, HID), lambda i: (0, 0)),
            pl.BlockSpec((F_IN, HID), lambda i: (0, 0)),
            pl.BlockSpec((HID, C_OUT), lambda i: (0, 0)),
        ],
        out_specs=[
            pl.BlockSpec((BLK, HID), lambda i: (i, 0)),
            pl.BlockSpec((BLK, C_OUT), lambda i: (i, 0)),
        ],
        out_shape=[
            jax.ShapeDtypeStruct((N, HID), jnp.float32),
            jax.ShapeDtypeStruct((N, C_OUT), jnp.float32),
        ],
    )(p1, x, W1_r, b1_l.reshape(1, HID), W1_l, W2_l)

    p2 = _segsum_c(g, srcg, dstg, zrows_c)

    out = pl.pallas_call(
        _layer2_body,
        grid=(GRID,),
        in_specs=[
            pl.BlockSpec((NC, BLK, C_OUT), lambda i: (0, i, 0)),
            pl.BlockSpec((BLK, HID), lambda i: (i, 0)),
            pl.BlockSpec((HID, C_OUT), lambda i: (0, 0)),
            pl.BlockSpec((1, C_OUT), lambda i: (0, 0)),
        ],
        out_specs=pl.BlockSpec((BLK, C_OUT), lambda i: (i, 0)),
        out_shape=jax.ShapeDtypeStruct((N, C_OUT), jnp.float32),
    )(p2, h, W2_r, b2_l.reshape(1, C_OUT))
    return out
